# baseline (speedup 1.0000x reference)
import functools

import jax
import jax.numpy as jnp
from jax import lax
from jax.experimental import pallas as pl
from jax.experimental.pallas import tpu as pltpu
from jax.experimental.pallas import tpu_sc as plsc

BATCH = 16384
LATENT = 64
HIDDEN = 32

NUM_CORES = 2
NUM_SUBCORES = 16
NUM_WORKERS = NUM_CORES * NUM_SUBCORES
ROWS_PER_W = BATCH // NUM_WORKERS
IDX_CHUNK = 128
NUM_CHUNKS = ROWS_PER_W // IDX_CHUNK

DENSE_BLOCK = 2048


@functools.cache
def _build_gather():
    mesh = plsc.VectorSubcoreMesh(core_axis_name="c", subcore_axis_name="s")

    @functools.partial(
        pl.kernel,
        mesh=mesh,
        compiler_params=pltpu.CompilerParams(use_tc_tiling_on_sc=False),
        out_type=jax.ShapeDtypeStruct((BATCH, LATENT), jnp.float32),
        scratch_types=[
            pltpu.VMEM((ROWS_PER_W,), jnp.int32),
            pltpu.VMEM((ROWS_PER_W, LATENT), jnp.float32),
            pltpu.SemaphoreType.DMA,
        ],
    )
    def gather(table_hbm, idx_hbm, out_hbm, idx_v, rows_v, sem):
        wid = lax.axis_index("s") * NUM_CORES + lax.axis_index("c")
        base = wid * ROWS_PER_W
        pltpu.sync_copy(idx_hbm.at[pl.ds(base, ROWS_PER_W)], idx_v)
        copies = [
            pltpu.make_async_copy(
                table_hbm.at[idx_v.at[pl.ds(j * IDX_CHUNK, IDX_CHUNK)]],
                rows_v.at[pl.ds(j * IDX_CHUNK, IDX_CHUNK)],
                sem,
            )
            for j in range(NUM_CHUNKS)
        ]
        for c in copies:
            c.start()
        for c in copies:
            c.wait()
        pltpu.sync_copy(rows_v, out_hbm.at[pl.ds(base, ROWS_PER_W)])

    return gather


def _dense_body(emb_ref, wt_ref, b_ref, out_ref):
    h = jnp.dot(emb_ref[...], wt_ref[...], preferred_element_type=jnp.float32)
    h = h + b_ref[...]
    out_ref[...] = jnp.sum(h * h, axis=1)


@functools.cache
def _build_dense():
    return pl.pallas_call(
        _dense_body,
        grid=(BATCH // DENSE_BLOCK,),
        in_specs=[
            pl.BlockSpec((DENSE_BLOCK, LATENT), lambda i: (i, 0)),
            pl.BlockSpec((LATENT, HIDDEN), lambda i: (0, 0)),
            pl.BlockSpec((1, HIDDEN), lambda i: (0, 0)),
        ],
        out_specs=pl.BlockSpec((DENSE_BLOCK,), lambda i: (i,)),
        out_shape=jax.ShapeDtypeStruct((BATCH,), jnp.float32),
    )


def kernel(user_ids, item_ids, user_table, item_table, W_user, b_user, W_item, b_item):
    del item_ids, item_table, W_item, b_item
    emb = _build_gather()(user_table, user_ids.astype(jnp.int32))
    return _build_dense()(emb, W_user.T, b_user.reshape(1, HIDDEN))
